# baseline (speedup 1.0000x reference)
import functools

import jax
import jax.numpy as jnp
from jax import lax
from jax.experimental import pallas as pl
from jax.experimental.pallas import tpu as pltpu
from jax.experimental.pallas import tpu_sc as plsc

N_STATE = 10000
E = 320000
D_IN = 16
D_H = 128
N = N_STATE + E
NT = 331776
NT_SUB = NT // 16
FS_ROWS = NT // 128
F16_ROWS = NT * 16 // 128
E_PAD = 327680
E_ROWS = E_PAD // 128
NCHUNK = 4
CHUNK = NT // NCHUNK
CH_SUB = CHUNK // 16
ACC_ROWS = CHUNK + 128
STEPS = 81
SROW = 32
FROW = 512

_MESH = dict(
    mesh=plsc.VectorSubcoreMesh(core_axis_name="c", subcore_axis_name="s"),
    compiler_params=pltpu.CompilerParams(use_tc_tiling_on_sc=False),
)


@functools.partial(
    pl.kernel,
    out_type=jax.ShapeDtypeStruct((2 * NT,), jnp.float32),
    scratch_types=[
        pltpu.VMEM((16, 128), jnp.int32),
        pltpu.VMEM((128,), jnp.float32),
        pltpu.VMEM((NT_SUB,), jnp.float32),
        pltpu.VMEM_SHARED((NT,), jnp.float32),
    ],
    **_MESH,
)
def _deg_kernel(dst_hbm, out_hbm, idx_d, ones_v, buf, acc_sh):
    c = lax.axis_index("c")
    s = lax.axis_index("s")
    for j in range(8):
        ones_v[pl.ds(j * 16, 16)] = jnp.ones((16,), jnp.float32)

    def _z(i, _):
        buf[pl.ds(i * 16, 16)] = jnp.zeros((16,), jnp.float32)
        return 0

    lax.fori_loop(0, NT_SUB // 16, _z, 0)
    pltpu.sync_copy(buf, acc_sh.at[pl.ds(s * NT_SUB, NT_SUB)])
    plsc.subcore_barrier()
    wid = s * 2 + c
    row0 = wid * (E_ROWS // 32)
    for sb in range(5):
        pltpu.sync_copy(dst_hbm.at[pl.ds(row0 + sb * 16, 16)], idx_d)
        for j in range(16):
            pltpu.sync_copy(ones_v, acc_sh.at[idx_d.at[j]], add=True)
    plsc.subcore_barrier()
    pltpu.sync_copy(acc_sh.at[pl.ds(s * NT_SUB, NT_SUB)], buf)
    pltpu.sync_copy(buf, out_hbm.at[pl.ds(c * NT + s * NT_SUB, NT_SUB)])


@functools.partial(
    pl.kernel,
    out_type=jax.ShapeDtypeStruct((NT, D_IN), jnp.float32),
    scratch_types=[
        pltpu.VMEM((8, 128), jnp.int32),
        pltpu.VMEM((8, 128), jnp.int32),
        pltpu.VMEM((1024, D_IN), jnp.float32),
        pltpu.VMEM((8, 128), jnp.int32),
        pltpu.VMEM((8, 128), jnp.int32),
        pltpu.VMEM((1024, D_IN), jnp.float32),
        pltpu.VMEM_SHARED((ACC_ROWS, D_IN), jnp.float32),
        pltpu.SemaphoreType.DMA,
        pltpu.SemaphoreType.DMA,
        pltpu.SemaphoreType.DMA,
        pltpu.SemaphoreType.DMA,
    ],
    **_MESH,
)
def _t1_kernel(y_hbm, src_hbm, dst_hbm, out_hbm,
               idx_s0, idx_d0, rows0, idx_s1, idx_d1, rows1, acc_sh,
               gsem0, ssem0, gsem1, ssem1):
    c = lax.axis_index("c")
    s = lax.axis_index("s")
    bufs = [(idx_s0, idx_d0, rows0, gsem0, ssem0),
            (idx_s1, idx_d1, rows1, gsem1, ssem1)]
    rows = rows0

    def _z(i, _):
        rows[i, :] = jnp.zeros((D_IN,), jnp.float32)
        return 0

    NSB = 20

    pieces = [(off, min(1024, CH_SUB - off)) for off in range(0, CH_SUB, 1024)]
    for ci in range(2):
        chunk = c * 2 + ci
        base = chunk * CHUNK
        lax.fori_loop(0, 1024, _z, 0)
        for (off, sz) in pieces:
            pltpu.sync_copy(rows.at[pl.ds(0, sz)],
                            acc_sh.at[pl.ds(s * CH_SUB + off, sz)])
        @pl.when(s == 0)
        def _():
            pltpu.sync_copy(rows.at[pl.ds(0, ACC_ROWS - CHUNK)],
                            acc_sh.at[pl.ds(CHUNK, ACC_ROWS - CHUNK)])

        plsc.subcore_barrier()
        row0 = s * (E_ROWS // 16)

        def _load_fire(sb, b):
            bidx_s, bidx_d, brows, bgsem, _ = b
            pltpu.sync_copy(src_hbm.at[pl.ds(row0 + sb * 8, 8)], bidx_s)
            pltpu.sync_copy(dst_hbm.at[pl.ds(row0 + sb * 8, 8)], bidx_d)
            for r in range(8):
                for k in range(8):
                    d = bidx_d[r, pl.ds(k * 16, 16)]
                    loc = d - base
                    ok = (loc >= 0) & (loc < CHUNK)
                    trash = CHUNK + lax.iota(jnp.int32, 16) + (k % 8) * 16
                    bidx_d[r, pl.ds(k * 16, 16)] = jnp.where(ok, loc, trash)
            return [
                pltpu.async_copy(y_hbm.at[bidx_s.at[j]],
                                 brows.at[pl.ds(j * 128, 128)], bgsem)
                for j in range(8)
            ]

        pend_g = [None, None]
        pend_s = [None, None]
        pend_g[0] = _load_fire(0, bufs[0])
        for sb in range(NSB):
            cur, nxt = sb % 2, (sb + 1) % 2
            bidx_s, bidx_d, brows, _, bssem = bufs[cur]
            for cp in pend_g[cur]:
                cp.wait()
            if pend_s[nxt] is not None:
                for cp in pend_s[nxt]:
                    cp.wait()
            if sb + 1 < NSB:
                pend_g[nxt] = _load_fire(sb + 1, bufs[nxt])
            pend_s[cur] = [
                pltpu.async_copy(brows.at[pl.ds(j * 128, 128)],
                                 acc_sh.at[bidx_d.at[j]], bssem, add=True)
                for j in range(8)
            ]
        for cp in pend_s[(NSB - 1) % 2]:
            cp.wait()
        plsc.subcore_barrier()
        for (off, sz) in pieces:
            pltpu.sync_copy(acc_sh.at[pl.ds(s * CH_SUB + off, sz)],
                            rows.at[pl.ds(0, sz)])
            pltpu.sync_copy(
                rows.at[pl.ds(0, sz)],
                out_hbm.at[pl.ds(base + s * CH_SUB + off, sz)])
        plsc.subcore_barrier()


@functools.partial(
    pl.kernel,
    out_type=jax.ShapeDtypeStruct((2 * NT,), jnp.float32),
    scratch_types=[
        pltpu.VMEM((16, 128), jnp.int32),
        pltpu.VMEM((16, 128), jnp.int32),
        pltpu.VMEM((2048,), jnp.float32),
        pltpu.VMEM((2048,), jnp.float32),
        pltpu.VMEM((NT_SUB,), jnp.float32),
        pltpu.VMEM_SHARED((NT,), jnp.float32),
        pltpu.SemaphoreType.DMA,
        pltpu.SemaphoreType.DMA,
        pltpu.SemaphoreType.DMA,
        pltpu.SemaphoreType.DMA,
    ],
    **_MESH,
)
def _t2_kernel(gv_hbm, src_hbm, dst_hbm, out_hbm,
               idx_s0, idx_d0, vals0, vals1, buf, acc_sh,
               gsem0, ssem0, gsem1, ssem1):
    c = lax.axis_index("c")
    s = lax.axis_index("s")
    bufs = [(idx_s0, vals0, gsem0, ssem0), (idx_d0, vals1, gsem1, ssem1)]

    def _z(i, _):
        buf[pl.ds(i * 16, 16)] = jnp.zeros((16,), jnp.float32)
        return 0

    lax.fori_loop(0, NT_SUB // 16, _z, 0)
    pltpu.sync_copy(buf, acc_sh.at[pl.ds(s * NT_SUB, NT_SUB)])
    plsc.subcore_barrier()
    wid = s * 2 + c
    row0 = wid * (E_ROWS // 32)
    NSB = 10

    def _load_fire(sb, b):
        bidx, bvals, bgsem, _ = b
        pltpu.sync_copy(src_hbm.at[pl.ds(row0 + sb * 8, 8)],
                        bidx.at[pl.ds(0, 8)])
        pltpu.sync_copy(dst_hbm.at[pl.ds(row0 + sb * 8, 8)],
                        bidx.at[pl.ds(8, 8)])
        return [
            pltpu.async_copy(gv_hbm.at[bidx.at[j]],
                             bvals.at[pl.ds(j * 128, 128)], bgsem)
            for j in range(8)
        ]

    pend_g = [None, None]
    pend_s = [None, None]
    pend_g[0] = _load_fire(0, bufs[0])
    for sb in range(NSB):
        cur, nxt = sb % 2, (sb + 1) % 2
        bidx, bvals, _, bssem = bufs[cur]
        for cp in pend_g[cur]:
            cp.wait()
        if pend_s[nxt] is not None:
            for cp in pend_s[nxt]:
                cp.wait()
        if sb + 1 < NSB:
            pend_g[nxt] = _load_fire(sb + 1, bufs[nxt])
        pend_s[cur] = [
            pltpu.async_copy(bvals.at[pl.ds(j * 128, 128)],
                             acc_sh.at[bidx.at[8 + j]], bssem, add=True)
            for j in range(8)
        ]
    for cp in pend_s[(NSB - 1) % 2]:
        cp.wait()
    plsc.subcore_barrier()
    pltpu.sync_copy(acc_sh.at[pl.ds(s * NT_SUB, NT_SUB)], buf)
    pltpu.sync_copy(buf, out_hbm.at[pl.ds(c * NT + s * NT_SUB, NT_SUB)])


def _prep_body(nft_ref, d0_ref, d1_ref, dinv_ref, yt_ref):
    deg = d0_ref[...] + d1_ref[...] + 1.0
    dv = lax.rsqrt(deg)
    dinv_ref[...] = dv
    yt_ref[...] = nft_ref[...] * dv


def _dense_body(t1t_ref, yt_ref, dinv_ref, w1_ref, b1_ref, w2_ref, w3_ref,
                gv_ref):
    dv = dinv_ref[...]
    z = t1t_ref[...] + yt_ref[...]
    h = lax.dot_general(w1_ref[...], z, (((0,), (0,)), ((), ())),
                        preferred_element_type=jnp.float32)
    x1 = jnp.maximum(h * dv + b1_ref[...], 0.0)
    w23 = jnp.dot(w2_ref[...], w3_ref[...], preferred_element_type=jnp.float32)
    gv_ref[...] = jnp.sum(x1 * w23, axis=0, keepdims=True) * dv


def _final_body(t2a_ref, t2b_ref, gv_ref, dinv_ref, b2_ref, w3_ref, b3_ref,
                out_ref):
    cc = jnp.dot(b2_ref[...], w3_ref[...],
                 preferred_element_type=jnp.float32) + b3_ref[...]
    out_ref[...] = (dinv_ref[...] * (t2a_ref[...] + t2b_ref[...] + gv_ref[...])
                    + cc[0, 0])


CSTEP = NT // STEPS


def _col(rows):
    return pl.BlockSpec((rows, CSTEP), lambda i: (0, i))


def _full_spec(shape):
    return pl.BlockSpec(shape, lambda i: tuple(0 for _ in shape))


def kernel(state, edge_index, edge_attr, W1, b1, W2, b2, W3, b3):
    f32 = jnp.float32
    src = edge_index[0]
    dst = edge_index[1]
    e_pad = jnp.arange(E, E_PAD, dtype=jnp.int32)
    src2d = jnp.concatenate([src, e_pad & 8191]).reshape(E_ROWS, 128)
    dst2d = jnp.concatenate([dst, N + (e_pad & 1023)]).reshape(E_ROWS, 128)
    nf_t = jnp.concatenate([state.reshape(-1, D_IN), edge_attr]).T

    degp = _deg_kernel(dst2d)
    d0 = degp[0:NT].reshape(1, NT)
    d1 = degp[NT:2 * NT].reshape(1, NT)

    dinv, y_t = pl.pallas_call(
        _prep_body,
        grid=(STEPS,),
        in_specs=[_col(D_IN), _col(1), _col(1)],
        out_specs=[_col(1), _col(D_IN)],
        out_shape=[
            jax.ShapeDtypeStruct((1, NT), f32),
            jax.ShapeDtypeStruct((D_IN, N), f32),
        ],
    )(nf_t, d0, d1)

    t1 = _t1_kernel(y_t.T, src2d, dst2d)

    gv = pl.pallas_call(
        _dense_body,
        grid=(STEPS,),
        in_specs=[
            _col(D_IN), _col(D_IN), _col(1),
            _full_spec((D_IN, D_H)), _full_spec((D_H, 1)),
            _full_spec((D_H, D_H)), _full_spec((D_H, 1)),
        ],
        out_specs=_col(1),
        out_shape=jax.ShapeDtypeStruct((1, NT), f32),
    )(t1.T, y_t, dinv, W1, b1.reshape(D_H, 1), W2, W3)

    t2p = _t2_kernel(gv.reshape(NT), src2d, dst2d)
    t2a = t2p[0:NT].reshape(1, NT)
    t2b = t2p[NT:2 * NT].reshape(1, NT)

    out_row = pl.pallas_call(
        _final_body,
        grid=(STEPS,),
        in_specs=[
            _col(1), _col(1), _col(1), _col(1),
            _full_spec((1, D_H)), _full_spec((D_H, 1)), _full_spec((1, 1)),
        ],
        out_specs=_col(1),
        out_shape=jax.ShapeDtypeStruct((1, NT), f32),
    )(t2a, t2b, gv, dinv, b2.reshape(1, D_H), W3, b3.reshape(1, 1))
    return out_row.reshape(NT)[0:N].reshape(N, 1)


# speedup vs baseline: 1.0075x; 1.0075x over previous
import functools

import jax
import jax.numpy as jnp
from jax import lax
from jax.experimental import pallas as pl
from jax.experimental.pallas import tpu as pltpu
from jax.experimental.pallas import tpu_sc as plsc

N_STATE = 10000
E = 320000
D_IN = 16
D_H = 128
N = N_STATE + E
NT = 331776
NT_SUB = NT // 16
FS_ROWS = NT // 128
F16_ROWS = NT * 16 // 128
E_PAD = 327680
E_ROWS = E_PAD // 128
NCHUNK = 4
CHUNK = NT // NCHUNK
CH_SUB = CHUNK // 16
ACC_ROWS = CHUNK + 128
STEPS = 81
SROW = 32
FROW = 512

_MESH = dict(
    mesh=plsc.VectorSubcoreMesh(core_axis_name="c", subcore_axis_name="s"),
    compiler_params=pltpu.CompilerParams(use_tc_tiling_on_sc=False),
)


@functools.partial(
    pl.kernel,
    out_type=jax.ShapeDtypeStruct((2 * NT,), jnp.float32),
    scratch_types=[
        pltpu.VMEM((16, 128), jnp.int32),
        pltpu.VMEM((128,), jnp.float32),
        pltpu.VMEM((NT_SUB,), jnp.float32),
        pltpu.VMEM_SHARED((NT,), jnp.float32),
    ],
    **_MESH,
)
def _deg_kernel(dst_hbm, out_hbm, idx_d, ones_v, buf, acc_sh):
    c = lax.axis_index("c")
    s = lax.axis_index("s")
    for j in range(8):
        ones_v[pl.ds(j * 16, 16)] = jnp.ones((16,), jnp.float32)

    def _z(i, _):
        buf[pl.ds(i * 16, 16)] = jnp.zeros((16,), jnp.float32)
        return 0

    lax.fori_loop(0, NT_SUB // 16, _z, 0)
    pltpu.sync_copy(buf, acc_sh.at[pl.ds(s * NT_SUB, NT_SUB)])
    plsc.subcore_barrier()
    wid = s * 2 + c
    row0 = wid * (E_ROWS // 32)
    for sb in range(5):
        pltpu.sync_copy(dst_hbm.at[pl.ds(row0 + sb * 16, 16)], idx_d)
        for j in range(16):
            pltpu.sync_copy(ones_v, acc_sh.at[idx_d.at[j]], add=True)
    plsc.subcore_barrier()
    pltpu.sync_copy(acc_sh.at[pl.ds(s * NT_SUB, NT_SUB)], buf)
    pltpu.sync_copy(buf, out_hbm.at[pl.ds(c * NT + s * NT_SUB, NT_SUB)])


@functools.partial(
    pl.kernel,
    out_type=jax.ShapeDtypeStruct((NT, D_IN), jnp.float32),
    scratch_types=[
        pltpu.VMEM((8, 128), jnp.int32),
        pltpu.VMEM((8, 128), jnp.int32),
        pltpu.VMEM((1024, D_IN), jnp.float32),
        pltpu.VMEM((8, 128), jnp.int32),
        pltpu.VMEM((8, 128), jnp.int32),
        pltpu.VMEM((1024, D_IN), jnp.float32),
        pltpu.VMEM_SHARED((ACC_ROWS, D_IN), jnp.float32),
        pltpu.SemaphoreType.DMA,
        pltpu.SemaphoreType.DMA,
        pltpu.SemaphoreType.DMA,
        pltpu.SemaphoreType.DMA,
    ],
    **_MESH,
)
def _t1_kernel(y_hbm, src_hbm, dst_hbm, out_hbm,
               idx_s0, idx_d0, rows0, idx_s1, idx_d1, rows1, acc_sh,
               gsem0, ssem0, gsem1, ssem1):
    c = lax.axis_index("c")
    s = lax.axis_index("s")
    bufs = [(idx_s0, idx_d0, rows0, gsem0, ssem0),
            (idx_s1, idx_d1, rows1, gsem1, ssem1)]
    rows = rows0

    def _z(i, _):
        rows[i, :] = jnp.zeros((D_IN,), jnp.float32)
        return 0

    NSB = 20

    pieces = [(off, min(1024, CH_SUB - off)) for off in range(0, CH_SUB, 1024)]
    for ci in range(2):
        chunk = c * 2 + ci
        base = chunk * CHUNK
        lax.fori_loop(0, 1024, _z, 0)
        for (off, sz) in pieces:
            pltpu.sync_copy(rows.at[pl.ds(0, sz)],
                            acc_sh.at[pl.ds(s * CH_SUB + off, sz)])
        @pl.when(s == 0)
        def _():
            pltpu.sync_copy(rows.at[pl.ds(0, ACC_ROWS - CHUNK)],
                            acc_sh.at[pl.ds(CHUNK, ACC_ROWS - CHUNK)])

        plsc.subcore_barrier()
        row0 = s * (E_ROWS // 16)

        def _load_fire(sb, b):
            bidx_s, bidx_d, brows, bgsem, _ = b
            pltpu.sync_copy(src_hbm.at[pl.ds(row0 + sb * 8, 8)], bidx_s)
            pltpu.sync_copy(dst_hbm.at[pl.ds(row0 + sb * 8, 8)], bidx_d)
            for r in range(8):
                for k in range(8):
                    d = bidx_d[r, pl.ds(k * 16, 16)]
                    loc = d - base
                    ok = (loc >= 0) & (loc < CHUNK)
                    trash = CHUNK + lax.iota(jnp.int32, 16) + (k % 8) * 16
                    bidx_d[r, pl.ds(k * 16, 16)] = jnp.where(ok, loc, trash)
            return [
                pltpu.async_copy(y_hbm.at[bidx_s.at[j]],
                                 brows.at[pl.ds(j * 128, 128)], bgsem)
                for j in range(8)
            ]

        pend_g = [None, None]
        pend_s = [None, None]
        pend_g[0] = _load_fire(0, bufs[0])
        for sb in range(NSB):
            cur, nxt = sb % 2, (sb + 1) % 2
            bidx_s, bidx_d, brows, _, bssem = bufs[cur]
            for cp in pend_g[cur]:
                cp.wait()
            if pend_s[nxt] is not None:
                for cp in pend_s[nxt]:
                    cp.wait()
            if sb + 1 < NSB:
                pend_g[nxt] = _load_fire(sb + 1, bufs[nxt])
            pend_s[cur] = [
                pltpu.async_copy(brows.at[pl.ds(j * 128, 128)],
                                 acc_sh.at[bidx_d.at[j]], bssem, add=True)
                for j in range(8)
            ]
        for cp in pend_s[(NSB - 1) % 2]:
            cp.wait()
        plsc.subcore_barrier()
        for (off, sz) in pieces:
            pltpu.sync_copy(acc_sh.at[pl.ds(s * CH_SUB + off, sz)],
                            rows.at[pl.ds(0, sz)])
            pltpu.sync_copy(
                rows.at[pl.ds(0, sz)],
                out_hbm.at[pl.ds(base + s * CH_SUB + off, sz)])
        plsc.subcore_barrier()


@functools.partial(
    pl.kernel,
    out_type=jax.ShapeDtypeStruct((2 * NT,), jnp.float32),
    scratch_types=[
        pltpu.VMEM((16, 128), jnp.int32),
        pltpu.VMEM((16, 128), jnp.int32),
        pltpu.VMEM((2048,), jnp.float32),
        pltpu.VMEM((NT_SUB,), jnp.float32),
        pltpu.VMEM_SHARED((NT,), jnp.float32),
        pltpu.SemaphoreType.DMA,
    ],
    **_MESH,
)
def _t2_kernel(gv_hbm, src_hbm, dst_hbm, out_hbm,
               idx_s, idx_d, vals, buf, acc_sh, sem):
    c = lax.axis_index("c")
    s = lax.axis_index("s")

    def _z(i, _):
        buf[pl.ds(i * 16, 16)] = jnp.zeros((16,), jnp.float32)
        return 0

    lax.fori_loop(0, NT_SUB // 16, _z, 0)
    pltpu.sync_copy(buf, acc_sh.at[pl.ds(s * NT_SUB, NT_SUB)])
    plsc.subcore_barrier()
    wid = s * 2 + c
    row0 = wid * (E_ROWS // 32)
    for sb in range(5):
        pltpu.sync_copy(src_hbm.at[pl.ds(row0 + sb * 16, 16)], idx_s)
        pltpu.sync_copy(dst_hbm.at[pl.ds(row0 + sb * 16, 16)], idx_d)
        cps = [
            pltpu.async_copy(gv_hbm.at[idx_s.at[j]],
                             vals.at[pl.ds(j * 128, 128)], sem)
            for j in range(16)
        ]
        for cp in cps:
            cp.wait()
        for j in range(16):
            pltpu.sync_copy(vals.at[pl.ds(j * 128, 128)],
                            acc_sh.at[idx_d.at[j]], add=True)
    plsc.subcore_barrier()
    pltpu.sync_copy(acc_sh.at[pl.ds(s * NT_SUB, NT_SUB)], buf)
    pltpu.sync_copy(buf, out_hbm.at[pl.ds(c * NT + s * NT_SUB, NT_SUB)])


def _prep_body(nft_ref, d0_ref, d1_ref, dinv_ref, yt_ref):
    deg = d0_ref[...] + d1_ref[...] + 1.0
    dv = lax.rsqrt(deg)
    dinv_ref[...] = dv
    yt_ref[...] = nft_ref[...] * dv


def _dense_body(t1t_ref, yt_ref, dinv_ref, w1_ref, b1_ref, w2_ref, w3_ref,
                gv_ref):
    dv = dinv_ref[...]
    z = t1t_ref[...] + yt_ref[...]
    h = lax.dot_general(w1_ref[...], z, (((0,), (0,)), ((), ())),
                        preferred_element_type=jnp.float32)
    x1 = jnp.maximum(h * dv + b1_ref[...], 0.0)
    w23 = jnp.dot(w2_ref[...], w3_ref[...], preferred_element_type=jnp.float32)
    gv_ref[...] = lax.dot_general(w23, x1, (((0,), (0,)), ((), ())),
                                  preferred_element_type=jnp.float32) * dv


def _final_body(t2a_ref, t2b_ref, gv_ref, dinv_ref, b2_ref, w3_ref, b3_ref,
                out_ref):
    cc = jnp.dot(b2_ref[...], w3_ref[...],
                 preferred_element_type=jnp.float32) + b3_ref[...]
    out_ref[...] = (dinv_ref[...] * (t2a_ref[...] + t2b_ref[...] + gv_ref[...])
                    + cc[0, 0])


CSTEP = NT // STEPS


def _col(rows):
    return pl.BlockSpec((rows, CSTEP), lambda i: (0, i))


def _full_spec(shape):
    return pl.BlockSpec(shape, lambda i: tuple(0 for _ in shape))


def kernel(state, edge_index, edge_attr, W1, b1, W2, b2, W3, b3):
    f32 = jnp.float32
    src = edge_index[0]
    dst = edge_index[1]
    e_pad = jnp.arange(E, E_PAD, dtype=jnp.int32)
    src2d = jnp.concatenate([src, e_pad & 8191]).reshape(E_ROWS, 128)
    dst2d = jnp.concatenate([dst, N + (e_pad & 1023)]).reshape(E_ROWS, 128)
    nf_t = jnp.concatenate([state.reshape(-1, D_IN), edge_attr]).T

    degp = _deg_kernel(dst2d)
    d0 = degp[0:NT].reshape(1, NT)
    d1 = degp[NT:2 * NT].reshape(1, NT)

    dinv, y_t = pl.pallas_call(
        _prep_body,
        grid=(STEPS,),
        in_specs=[_col(D_IN), _col(1), _col(1)],
        out_specs=[_col(1), _col(D_IN)],
        out_shape=[
            jax.ShapeDtypeStruct((1, NT), f32),
            jax.ShapeDtypeStruct((D_IN, N), f32),
        ],
    )(nf_t, d0, d1)

    t1 = _t1_kernel(y_t.T, src2d, dst2d)

    gv = pl.pallas_call(
        _dense_body,
        grid=(STEPS,),
        in_specs=[
            _col(D_IN), _col(D_IN), _col(1),
            _full_spec((D_IN, D_H)), _full_spec((D_H, 1)),
            _full_spec((D_H, D_H)), _full_spec((D_H, 1)),
        ],
        out_specs=_col(1),
        out_shape=jax.ShapeDtypeStruct((1, NT), f32),
    )(t1.T, y_t, dinv, W1, b1.reshape(D_H, 1), W2, W3)

    t2p = _t2_kernel(gv.reshape(NT), src2d, dst2d)
    t2a = t2p[0:NT].reshape(1, NT)
    t2b = t2p[NT:2 * NT].reshape(1, NT)

    out_row = pl.pallas_call(
        _final_body,
        grid=(STEPS,),
        in_specs=[
            _col(1), _col(1), _col(1), _col(1),
            _full_spec((1, D_H)), _full_spec((D_H, 1)), _full_spec((1, 1)),
        ],
        out_specs=_col(1),
        out_shape=jax.ShapeDtypeStruct((1, NT), f32),
    )(t2a, t2b, gv, dinv, b2.reshape(1, D_H), W3, b3.reshape(1, 1))
    return out_row.reshape(NT)[0:N].reshape(N, 1)
